# baseline (speedup 1.0000x reference)
import jax
import jax.numpy as jnp
from jax.experimental import pallas as pl

N = 10000
D = 128
E = 320000
BLK = 400


def _make_tc_layer(relu):
    def tc_body(h_ref, agg_ref, deg_ref, ws_ref, wn_ref, b_ref, o_ref):
        recip = 1.0 / jnp.maximum(deg_ref[:, 0:1], 1.0)
        hn = agg_ref[...] * recip
        out = (jnp.dot(h_ref[...], ws_ref[...],
                       preferred_element_type=jnp.float32)
               + jnp.dot(hn, wn_ref[...],
                         preferred_element_type=jnp.float32)
               + b_ref[...])
        o_ref[...] = jnp.maximum(out, 0.0) if relu else out

    return pl.pallas_call(
        tc_body,
        grid=(N // BLK,),
        in_specs=[
            pl.BlockSpec((BLK, D), lambda i: (i, 0)),
            pl.BlockSpec((BLK, D), lambda i: (i, 0)),
            pl.BlockSpec((BLK, 8), lambda i: (i, 0)),
            pl.BlockSpec((D, D), lambda i: (0, 0)),
            pl.BlockSpec((D, D), lambda i: (0, 0)),
            pl.BlockSpec((1, D), lambda i: (0, 0)),
        ],
        out_specs=pl.BlockSpec((BLK, D), lambda i: (i, 0)),
        out_shape=jax.ShapeDtypeStruct((N, D), jnp.float32),
    )


def kernel(x, edge_index, W_self1, W_neigh1, b1, W_self2, W_neigh2, b2):
    src = edge_index[0].astype(jnp.int32)
    dst = edge_index[1].astype(jnp.int32)
    key = jnp.sort(dst * 16384 + src)
    dst_s = key >> 14
    src_s = key & 16383
    bounds = jnp.searchsorted(dst_s, jnp.arange(N + 1, dtype=jnp.int32))
    deg = (bounds[1:] - bounds[:-1]).astype(jnp.float32)
    deg8 = jnp.broadcast_to(deg[:, None], (N, 8))

    tc1 = _make_tc_layer(True)
    tc2 = _make_tc_layer(False)

    agg1 = jax.ops.segment_sum(jnp.take(x, src_s, axis=0, mode="clip"),
                               dst_s, num_segments=N,
                               indices_are_sorted=True)
    h1 = tc1(x, agg1, deg8, W_self1, W_neigh1, b1.reshape(1, D))

    agg2 = jax.ops.segment_sum(jnp.take(h1, src_s, axis=0, mode="clip"),
                               dst_s, num_segments=N,
                               indices_are_sorted=True)
    h2 = tc2(h1, agg2, deg8, W_self2, W_neigh2, b2.reshape(1, D))
    return (x, h1, h2)
